# baseline (speedup 1.0000x reference)
import jax
import jax.numpy as jnp
from jax import lax
from jax.experimental import pallas as pl
from jax.experimental.pallas import tpu as pltpu
from jax.experimental.pallas import tpu_sc as plsc

_NUM_CORES = 2
_NUM_SUBCORES = 16
_NUM_WORKERS = _NUM_CORES * _NUM_SUBCORES

_BATCH = 16384
_DIM = 128
_B_PER_W = _BATCH // _NUM_WORKERS


_CHUNK = 64
_NCHUNKS = _B_PER_W // _CHUNK
_NBUF = 3
_HBM_CHUNKS = 2
_TABLE_ROWS = 1000
_STAGE_ROWS = 64


def _gather_body(table_hbm, idx_hbm, out_hbm, table_sh, idx_v, rows_v,
                 gsem0, gsem1, ssem0, ssem1, ssem2, stsem):
    sid = lax.axis_index("s")
    wid = sid * _NUM_CORES + lax.axis_index("c")
    base = wid * _B_PER_W
    stage = lax.min(sid * _STAGE_ROWS, _TABLE_ROWS - _STAGE_ROWS)
    stage_cp = pltpu.async_copy(table_hbm.at[pl.ds(stage, _STAGE_ROWS)],
                                table_sh.at[pl.ds(stage, _STAGE_ROWS)], stsem)
    idx_cp = pltpu.async_copy(idx_hbm.at[pl.ds(base, _B_PER_W)], idx_v, gsem1)
    idx_cp.wait()
    gsems = (gsem0, gsem1)
    ssems = (ssem0, ssem1, ssem2)
    gathers = [None] * _NCHUNKS
    writes = [None] * _NCHUNKS

    def start_gather(c):
        src = table_hbm if c < _HBM_CHUNKS else table_sh
        gathers[c] = pltpu.async_copy(
            src.at[idx_v.at[pl.ds(c * _CHUNK, _CHUNK)]],
            rows_v.at[c % _NBUF],
            gsems[c % 2],
        )

    def start_write(c):
        writes[c] = pltpu.async_copy(
            rows_v.at[c % _NBUF],
            out_hbm.at[pl.ds(base + c * _CHUNK, _CHUNK)],
            ssems[c % _NBUF],
        )

    start_gather(0)
    for c in range(1, _NCHUNKS):
        if c == _HBM_CHUNKS:
            stage_cp.wait()
            plsc.subcore_barrier()
        if c >= _NBUF:
            writes[c - _NBUF].wait()
        start_gather(c)
        gathers[c - 1].wait()
        start_write(c - 1)
    gathers[_NCHUNKS - 1].wait()
    start_write(_NCHUNKS - 1)
    for c in range(_NCHUNKS - _NBUF, _NCHUNKS):
        writes[c].wait()


@jax.jit
def kernel(t, pos_encoding):
    idx = t.reshape(_BATCH).astype(jnp.int32)
    mesh = plsc.VectorSubcoreMesh(
        core_axis_name="c",
        subcore_axis_name="s",
        num_cores=_NUM_CORES,
        num_subcores=_NUM_SUBCORES,
    )
    gather = pl.kernel(
        _gather_body,
        out_type=jax.ShapeDtypeStruct((_BATCH, _DIM), jnp.float32),
        mesh=mesh,
        scratch_types=[
            pltpu.VMEM_SHARED((_TABLE_ROWS, _DIM), jnp.float32),
            pltpu.VMEM((_B_PER_W,), jnp.int32),
            pltpu.VMEM((_NBUF, _CHUNK, _DIM), jnp.float32),
            pltpu.SemaphoreType.DMA,
            pltpu.SemaphoreType.DMA,
            pltpu.SemaphoreType.DMA,
            pltpu.SemaphoreType.DMA,
            pltpu.SemaphoreType.DMA,
            pltpu.SemaphoreType.DMA,
        ],
    )
    return gather(pos_encoding, idx)


# speedup vs baseline: 1.0255x; 1.0255x over previous
import jax
import jax.numpy as jnp
from jax import lax
from jax.experimental import pallas as pl
from jax.experimental.pallas import tpu as pltpu
from jax.experimental.pallas import tpu_sc as plsc

_NUM_CORES = 2
_NUM_SUBCORES = 16
_NUM_WORKERS = _NUM_CORES * _NUM_SUBCORES

_BATCH = 16384
_DIM = 128
_B_PER_W = _BATCH // _NUM_WORKERS


_CHUNK = 64
_NCHUNKS = _B_PER_W // _CHUNK
_NBUF = 4
_HBM_CHUNKS = 0
_TABLE_ROWS = 1000
_STAGE_ROWS = 64


def _gather_body(table_hbm, idx_hbm, out_hbm, table_sh, idx_v, rows_v,
                 gsem0, gsem1, ssem0, ssem1, ssem2, ssem3, stsem):
    sid = lax.axis_index("s")
    wid = sid * _NUM_CORES + lax.axis_index("c")
    base = wid * _B_PER_W
    stage = lax.min(sid * _STAGE_ROWS, _TABLE_ROWS - _STAGE_ROWS)
    stage_cp = pltpu.async_copy(table_hbm.at[pl.ds(stage, _STAGE_ROWS)],
                                table_sh.at[pl.ds(stage, _STAGE_ROWS)], stsem)
    idx_cp = pltpu.async_copy(idx_hbm.at[pl.ds(base, _B_PER_W)], idx_v, gsem1)
    idx_cp.wait()
    if _HBM_CHUNKS == 0:
        stage_cp.wait()
        plsc.subcore_barrier()
    gsems = (gsem0, gsem1)
    ssems = (ssem0, ssem1, ssem2, ssem3)[:_NBUF]
    gathers = [None] * _NCHUNKS
    writes = [None] * _NCHUNKS

    def start_gather(c):
        src = table_hbm if c < _HBM_CHUNKS else table_sh
        gathers[c] = pltpu.async_copy(
            src.at[idx_v.at[pl.ds(c * _CHUNK, _CHUNK)]],
            rows_v.at[c % _NBUF],
            gsems[c % 2],
        )

    def start_write(c):
        writes[c] = pltpu.async_copy(
            rows_v.at[c % _NBUF],
            out_hbm.at[pl.ds(base + c * _CHUNK, _CHUNK)],
            ssems[c % _NBUF],
        )

    start_gather(0)
    for c in range(1, _NCHUNKS):
        if c == _HBM_CHUNKS and _HBM_CHUNKS > 0:
            stage_cp.wait()
            plsc.subcore_barrier()
        if c >= _NBUF:
            writes[c - _NBUF].wait()
        start_gather(c)
        gathers[c - 1].wait()
        start_write(c - 1)
    gathers[_NCHUNKS - 1].wait()
    start_write(_NCHUNKS - 1)
    for c in range(_NCHUNKS - _NBUF, _NCHUNKS):
        writes[c].wait()


@jax.jit
def kernel(t, pos_encoding):
    idx = t.reshape(_BATCH).astype(jnp.int32)
    mesh = plsc.VectorSubcoreMesh(
        core_axis_name="c",
        subcore_axis_name="s",
        num_cores=_NUM_CORES,
        num_subcores=_NUM_SUBCORES,
    )
    gather = pl.kernel(
        _gather_body,
        out_type=jax.ShapeDtypeStruct((_BATCH, _DIM), jnp.float32),
        mesh=mesh,
        scratch_types=[
            pltpu.VMEM_SHARED((_TABLE_ROWS, _DIM), jnp.float32),
            pltpu.VMEM((_B_PER_W,), jnp.int32),
            pltpu.VMEM((_NBUF, _CHUNK, _DIM), jnp.float32),
            pltpu.SemaphoreType.DMA,
            pltpu.SemaphoreType.DMA,
            pltpu.SemaphoreType.DMA,
            pltpu.SemaphoreType.DMA,
            pltpu.SemaphoreType.DMA,
            pltpu.SemaphoreType.DMA,
            pltpu.SemaphoreType.DMA,
        ],
    )
    return gather(pos_encoding, idx)


# speedup vs baseline: 1.0502x; 1.0240x over previous
import jax
import jax.numpy as jnp
from jax import lax
from jax.experimental import pallas as pl
from jax.experimental.pallas import tpu as pltpu
from jax.experimental.pallas import tpu_sc as plsc

_NUM_CORES = 2
_NUM_SUBCORES = 16
_NUM_WORKERS = _NUM_CORES * _NUM_SUBCORES

_BATCH = 16384
_DIM = 128
_B_PER_W = _BATCH // _NUM_WORKERS


_CHUNK = 128
_NCHUNKS = _B_PER_W // _CHUNK
_NBUF = 3
_HBM_CHUNKS = 0
_TABLE_ROWS = 1000
_STAGE_ROWS = 64


def _gather_body(table_hbm, idx_hbm, out_hbm, table_sh, idx_v, rows_v,
                 gsem0, gsem1, ssem0, ssem1, ssem2, ssem3, stsem):
    sid = lax.axis_index("s")
    wid = sid * _NUM_CORES + lax.axis_index("c")
    base = wid * _B_PER_W
    stage = lax.min(sid * _STAGE_ROWS, _TABLE_ROWS - _STAGE_ROWS)
    stage_cp = pltpu.async_copy(table_hbm.at[pl.ds(stage, _STAGE_ROWS)],
                                table_sh.at[pl.ds(stage, _STAGE_ROWS)], stsem)
    idx_cp = pltpu.async_copy(idx_hbm.at[pl.ds(base, _B_PER_W)], idx_v, gsem1)
    idx_cp.wait()
    if _HBM_CHUNKS == 0:
        stage_cp.wait()
        plsc.subcore_barrier()
    gsems = (gsem0, gsem1)
    ssems = (ssem0, ssem1, ssem2, ssem3)[:_NBUF]
    gathers = [None] * _NCHUNKS
    writes = [None] * _NCHUNKS

    def start_gather(c):
        src = table_hbm if c < _HBM_CHUNKS else table_sh
        gathers[c] = pltpu.async_copy(
            src.at[idx_v.at[pl.ds(c * _CHUNK, _CHUNK)]],
            rows_v.at[c % _NBUF],
            gsems[c % 2],
        )

    def start_write(c):
        writes[c] = pltpu.async_copy(
            rows_v.at[c % _NBUF],
            out_hbm.at[pl.ds(base + c * _CHUNK, _CHUNK)],
            ssems[c % _NBUF],
        )

    start_gather(0)
    for c in range(1, _NCHUNKS):
        if c == _HBM_CHUNKS and _HBM_CHUNKS > 0:
            stage_cp.wait()
            plsc.subcore_barrier()
        if c >= _NBUF:
            writes[c - _NBUF].wait()
        start_gather(c)
        gathers[c - 1].wait()
        start_write(c - 1)
    gathers[_NCHUNKS - 1].wait()
    start_write(_NCHUNKS - 1)
    for c in range(_NCHUNKS - _NBUF, _NCHUNKS):
        writes[c].wait()


@jax.jit
def kernel(t, pos_encoding):
    idx = t.reshape(_BATCH).astype(jnp.int32)
    mesh = plsc.VectorSubcoreMesh(
        core_axis_name="c",
        subcore_axis_name="s",
        num_cores=_NUM_CORES,
        num_subcores=_NUM_SUBCORES,
    )
    gather = pl.kernel(
        _gather_body,
        out_type=jax.ShapeDtypeStruct((_BATCH, _DIM), jnp.float32),
        mesh=mesh,
        scratch_types=[
            pltpu.VMEM_SHARED((_TABLE_ROWS, _DIM), jnp.float32),
            pltpu.VMEM((_B_PER_W,), jnp.int32),
            pltpu.VMEM((_NBUF, _CHUNK, _DIM), jnp.float32),
            pltpu.SemaphoreType.DMA,
            pltpu.SemaphoreType.DMA,
            pltpu.SemaphoreType.DMA,
            pltpu.SemaphoreType.DMA,
            pltpu.SemaphoreType.DMA,
            pltpu.SemaphoreType.DMA,
            pltpu.SemaphoreType.DMA,
        ],
    )
    return gather(pos_encoding, idx)
